# baseline (speedup 1.0000x reference)
import functools

import jax
import jax.numpy as jnp
from jax import lax
from jax.experimental import pallas as pl
from jax.experimental.pallas import tpu as pltpu
from jax.experimental.pallas import tpu_sc as plsc

SIZE = 0.4
N = 1_000_000
H = N // 2
NWORKERS = 32
TILE = 2000
NTILES = H // TILE
GROUPS = TILE // 16
MAXK = 8


UNROLL = 5


def _compute_tile(xb, yb, zb, out_ref):
    half = jnp.float32(0.5)
    three_half = jnp.float32(1.5)
    magic = jnp.int32(0x5F3759DF)

    def one(o):
        x = xb[pl.ds(o, 16)]
        y = yb[pl.ds(o, 16)]
        z = zb[pl.ds(o, 16)]
        qx = jnp.abs(x) - SIZE
        qy = jnp.abs(y) - SIZE
        qz = jnp.abs(z) - SIZE
        m = jnp.maximum(jnp.maximum(qx, qy), qz)
        rx = jnp.maximum(qx, 0.0)
        ry = jnp.maximum(qy, 0.0)
        rz = jnp.maximum(qz, 0.0)
        v = rx * rx + ry * ry + rz * rz
        i = magic - (lax.bitcast_convert_type(v, jnp.int32) >> 1)
        y0 = lax.bitcast_convert_type(i, jnp.float32)
        hv = half * v
        y1 = y0 * (three_half - hv * y0 * y0)
        y2 = y1 * (three_half - hv * y1 * y1)
        out_ref[pl.ds(o, 16)] = v * y2 - jnp.minimum(m, 0.0)

    def body(g, carry):
        base = g * (16 * UNROLL)
        for u in range(UNROLL):
            one(base + u * 16)
        return carry

    lax.fori_loop(0, GROUPS // UNROLL, body, 0)


def _sc_kernel(x_hbm, y_hbm, z_hbm, out_hbm,
               xb0, yb0, zb0, xb1, yb1, zb1, ob0, ob1,
               isem0, isem1, osem0, osem1):
    wid = lax.axis_index("s") * 2 + lax.axis_index("c")
    ins = ((xb0, yb0, zb0), (xb1, yb1, zb1))
    obs = (ob0, ob1)
    isems = (isem0, isem1)
    osems = (osem0, osem1)
    hbms = (x_hbm, y_hbm, z_hbm)

    def start_in(t, b):
        for h, buf in zip(hbms, ins[b]):
            pltpu.async_copy(h.at[pl.ds(t * TILE, TILE)], buf, isems[b])

    def wait_in(t, b):
        for h, buf in zip(hbms, ins[b]):
            pltpu.make_async_copy(h.at[pl.ds(t * TILE, TILE)], buf,
                                  isems[b]).wait()

    def out_slice(t):
        return out_hbm.at[pl.ds(t * TILE, TILE)]

    for b in range(2):
        t = wid + b * NWORKERS

        @pl.when(t < NTILES)
        def _():
            start_in(t, b)

    def outer(k, carry):
        kk = k * 2
        for b in range(2):
            t = wid + (kk + b) * NWORKERS

            @pl.when(t < NTILES)
            def _():
                @pl.when(kk + b >= 2)
                def _():
                    pltpu.make_async_copy(obs[b], out_slice(t), osems[b]).wait()

                wait_in(t, b)
                _compute_tile(*ins[b], obs[b])
                pltpu.async_copy(obs[b], out_slice(t), osems[b])

            t2 = wid + (kk + b + 2) * NWORKERS

            @pl.when(t2 < NTILES)
            def _():
                start_in(t2, b)

        return carry

    lax.fori_loop(0, MAXK // 2, outer, 0)

    for b in range(2):
        pltpu.make_async_copy(obs[b], out_slice(wid), osems[b]).wait()


@jax.jit
def _run(xs, ys, zs):
    mesh = plsc.VectorSubcoreMesh(core_axis_name="c", subcore_axis_name="s")
    f = functools.partial(
        pl.kernel,
        mesh=mesh,
        compiler_params=pltpu.CompilerParams(needs_layout_passes=False),
        out_type=jax.ShapeDtypeStruct((H,), jnp.float32),
        scratch_types=[
            pltpu.VMEM((TILE,), jnp.float32),
            pltpu.VMEM((TILE,), jnp.float32),
            pltpu.VMEM((TILE,), jnp.float32),
            pltpu.VMEM((TILE,), jnp.float32),
            pltpu.VMEM((TILE,), jnp.float32),
            pltpu.VMEM((TILE,), jnp.float32),
            pltpu.VMEM((TILE,), jnp.float32),
            pltpu.VMEM((TILE,), jnp.float32),
            pltpu.SemaphoreType.DMA,
            pltpu.SemaphoreType.DMA,
            pltpu.SemaphoreType.DMA,
            pltpu.SemaphoreType.DMA,
        ],
    )(_sc_kernel)
    return f(xs, ys, zs)


def kernel(inputs):
    lo = inputs[:H]
    hi = inputs[H:]
    out0 = _run(lo[:, 0], lo[:, 1], lo[:, 2])
    out1 = _run(hi[:, 0], hi[:, 1], hi[:, 2])
    return jnp.concatenate([out0, out1])


# speedup vs baseline: 1.1468x; 1.1468x over previous
import functools

import jax
import jax.numpy as jnp
from jax import lax
from jax.experimental import pallas as pl
from jax.experimental.pallas import tpu as pltpu
from jax.experimental.pallas import tpu_sc as plsc

SIZE = 0.4
N = 1_000_000
NWORKERS = 32
TILE = 4000
NTILES = N // TILE
GROUPS = TILE // 16
MAXK = 8


UNROLL = 5


def _compute_tile(xb, yb, zb, out_ref):
    half = jnp.float32(0.5)
    three_half = jnp.float32(1.5)
    magic = jnp.int32(0x5F3759DF)

    def one(o):
        x = xb[pl.ds(o, 16)]
        y = yb[pl.ds(o, 16)]
        z = zb[pl.ds(o, 16)]
        qx = jnp.abs(x) - SIZE
        qy = jnp.abs(y) - SIZE
        qz = jnp.abs(z) - SIZE
        m = jnp.maximum(jnp.maximum(qx, qy), qz)
        rx = jnp.maximum(qx, 0.0)
        ry = jnp.maximum(qy, 0.0)
        rz = jnp.maximum(qz, 0.0)
        v = rx * rx + ry * ry + rz * rz
        i = magic - (lax.bitcast_convert_type(v, jnp.int32) >> 1)
        y0 = lax.bitcast_convert_type(i, jnp.float32)
        hv = half * v
        y1 = y0 * (three_half - hv * y0 * y0)
        y2 = y1 * (three_half - hv * y1 * y1)
        out_ref[pl.ds(o, 16)] = v * y2 - jnp.minimum(m, 0.0)

    def body(g, carry):
        base = g * (16 * UNROLL)
        for u in range(UNROLL):
            one(base + u * 16)
        return carry

    lax.fori_loop(0, GROUPS // UNROLL, body, 0)


def _sc_kernel(x_hbm, y_hbm, z_hbm, out_hbm,
               xb0, yb0, zb0, xb1, yb1, zb1, ob0, ob1,
               isem0, isem1, osem0, osem1):
    wid = lax.axis_index("s") * 2 + lax.axis_index("c")
    ins = ((xb0, yb0, zb0), (xb1, yb1, zb1))
    obs = (ob0, ob1)
    isems = (isem0, isem1)
    osems = (osem0, osem1)
    hbms = (x_hbm, y_hbm, z_hbm)

    def start_in(t, b):
        for h, buf in zip(hbms, ins[b]):
            pltpu.async_copy(h.at[pl.ds(t * TILE, TILE)], buf, isems[b])

    def wait_in(t, b):
        for h, buf in zip(hbms, ins[b]):
            pltpu.make_async_copy(h.at[pl.ds(t * TILE, TILE)], buf,
                                  isems[b]).wait()

    def out_slice(t):
        return out_hbm.at[pl.ds(t * TILE, TILE)]

    for b in range(2):
        t = wid + b * NWORKERS

        @pl.when(t < NTILES)
        def _():
            start_in(t, b)

    def outer(k, carry):
        kk = k * 2
        for b in range(2):
            t = wid + (kk + b) * NWORKERS

            @pl.when(t < NTILES)
            def _():
                @pl.when(kk + b >= 2)
                def _():
                    pltpu.make_async_copy(obs[b], out_slice(t), osems[b]).wait()

                wait_in(t, b)
                _compute_tile(*ins[b], obs[b])
                pltpu.async_copy(obs[b], out_slice(t), osems[b])

            t2 = wid + (kk + b + 2) * NWORKERS

            @pl.when(t2 < NTILES)
            def _():
                start_in(t2, b)

        return carry

    lax.fori_loop(0, MAXK // 2, outer, 0)

    for b in range(2):
        pltpu.make_async_copy(obs[b], out_slice(wid), osems[b]).wait()


@jax.jit
def _run(xs, ys, zs):
    mesh = plsc.VectorSubcoreMesh(core_axis_name="c", subcore_axis_name="s")
    f = functools.partial(
        pl.kernel,
        mesh=mesh,
        compiler_params=pltpu.CompilerParams(needs_layout_passes=False),
        out_type=jax.ShapeDtypeStruct((N,), jnp.float32),
        scratch_types=[
            pltpu.VMEM((TILE,), jnp.float32),
            pltpu.VMEM((TILE,), jnp.float32),
            pltpu.VMEM((TILE,), jnp.float32),
            pltpu.VMEM((TILE,), jnp.float32),
            pltpu.VMEM((TILE,), jnp.float32),
            pltpu.VMEM((TILE,), jnp.float32),
            pltpu.VMEM((TILE,), jnp.float32),
            pltpu.VMEM((TILE,), jnp.float32),
            pltpu.SemaphoreType.DMA,
            pltpu.SemaphoreType.DMA,
            pltpu.SemaphoreType.DMA,
            pltpu.SemaphoreType.DMA,
        ],
    )(_sc_kernel)
    return f(xs, ys, zs)


def kernel(inputs):
    return _run(inputs[:, 0], inputs[:, 1], inputs[:, 2])
